# baseline (speedup 1.0000x reference)
import functools
import jax
import jax.numpy as jnp
from jax import lax
from jax.experimental import pallas as pl
from jax.experimental.pallas import tpu as pltpu
from jax.experimental.pallas import tpu_sc as plsc

V, D, B = 8192, 1024, 4096
_info = plsc.get_sparse_core_info()
NC, NS = _info.num_cores, _info.num_subcores
NW = NC * NS
B_PER_W = B // NW


def _gather_kernel(table_hbm, idx_hbm, out_hbm, idx_v, sem):
    wid = lax.axis_index("s") * NC + lax.axis_index("c")
    base = wid * B_PER_W
    pltpu.sync_copy(idx_hbm.at[pl.ds(base, B_PER_W)], idx_v)
    for j in range(B_PER_W // 16):
        v = idx_v[pl.ds(j * 16, 16)]
        for k in range(16):
            r = v[k]
            pltpu.async_copy(
                table_hbm.at[pl.ds(r, 1)],
                out_hbm.at[pl.ds(base + j * 16 + k, 1)], sem)
    pltpu.make_async_copy(
        table_hbm.at[pl.ds(0, B_PER_W)], out_hbm.at[pl.ds(base, B_PER_W)],
        sem).wait()


@jax.jit
def _gather(table, idx):
    k = functools.partial(
        pl.kernel,
        mesh=plsc.VectorSubcoreMesh(core_axis_name="c", subcore_axis_name="s"),
        out_type=jax.ShapeDtypeStruct((B, D), jnp.float32),
        scratch_types=[
            pltpu.VMEM((B_PER_W,), jnp.int32),
            pltpu.SemaphoreType.DMA,
        ],
    )(_gather_kernel)
    return k(table, idx)


def kernel(hidden_state, word_indices):
    table = hidden_state.reshape(V, D)
    idx = word_indices.astype(jnp.int32)
    out = _gather(table, idx)
    return out.reshape(1, B, D)


# speedup vs baseline: 15.8140x; 15.8140x over previous
import functools

import jax
import jax.numpy as jnp
from jax import lax
from jax.experimental import pallas as pl
from jax.experimental.pallas import tpu as pltpu
from jax.experimental.pallas import tpu_sc as plsc

V, D, B = 8192, 1024, 4096
_info = plsc.get_sparse_core_info()
NC, NS = _info.num_cores, _info.num_subcores
NW = NC * NS
B_PER_W = B // NW
CH = 32
NCHUNK = B_PER_W // CH


def _gather_kernel(table_hbm, idx_hbm, out_hbm, idx_v, buf0, buf1, gsem0,
                   gsem1, osem0, osem1):
    wid = lax.axis_index("s") * NC + lax.axis_index("c")
    base = wid * B_PER_W
    pltpu.sync_copy(idx_hbm.at[pl.ds(base, B_PER_W)], idx_v)

    bufs = (buf0, buf1)
    gsems = (gsem0, gsem1)
    osems = (osem0, osem1)

    gathers = [None] * NCHUNK
    outs = [None] * NCHUNK
    gathers[0] = pltpu.async_copy(
        table_hbm.at[idx_v.at[pl.ds(0, CH)]], bufs[0], gsems[0])
    for i in range(NCHUNK):
        b = i % 2
        gathers[i].wait()
        outs[i] = pltpu.async_copy(
            bufs[b], out_hbm.at[pl.ds(base + i * CH, CH)], osems[b])
        if i + 1 < NCHUNK:
            nb = (i + 1) % 2
            if i >= 1:
                outs[i - 1].wait()
            gathers[i + 1] = pltpu.async_copy(
                table_hbm.at[idx_v.at[pl.ds((i + 1) * CH, CH)]],
                bufs[nb], gsems[nb])
    outs[NCHUNK - 2].wait()
    outs[NCHUNK - 1].wait()


@jax.jit
def _gather(table, idx):
    k = functools.partial(
        pl.kernel,
        mesh=plsc.VectorSubcoreMesh(core_axis_name="c", subcore_axis_name="s"),
        out_type=jax.ShapeDtypeStruct((B, D), jnp.float32),
        scratch_types=[
            pltpu.VMEM((B_PER_W,), jnp.int32),
            pltpu.VMEM((CH, D), jnp.float32),
            pltpu.VMEM((CH, D), jnp.float32),
            pltpu.SemaphoreType.DMA,
            pltpu.SemaphoreType.DMA,
            pltpu.SemaphoreType.DMA,
            pltpu.SemaphoreType.DMA,
        ],
    )(_gather_kernel)
    return k(table, idx)


def kernel(hidden_state, word_indices):
    table = hidden_state.reshape(V, D)
    idx = word_indices.astype(jnp.int32)
    out = _gather(table, idx)
    return out.reshape(1, B, D)


# speedup vs baseline: 15.9163x; 1.0065x over previous
import functools

import jax
import jax.numpy as jnp
from jax import lax
from jax.experimental import pallas as pl
from jax.experimental.pallas import tpu as pltpu
from jax.experimental.pallas import tpu_sc as plsc

V, D, B = 8192, 1024, 4096
_info = plsc.get_sparse_core_info()
NC, NS = _info.num_cores, _info.num_subcores
NW = NC * NS
B_PER_W = B // NW
CH = 16
NCHUNK = B_PER_W // CH


def _gather_kernel(table_hbm, idx_hbm, out_hbm, idx_v, vb0, vb1, sb0, sb1,
                   gsem0, gsem1, csem0, csem1, osem0, osem1):
    cid = lax.axis_index("c")
    sid = lax.axis_index("s")
    wid = sid * NC + cid
    base = wid * B_PER_W
    pltpu.sync_copy(idx_hbm.at[pl.ds(base, B_PER_W)], idx_v)

    vbufs = (vb0, vb1)
    sbufs = (sb0, sb1)
    gsems = (gsem0, gsem1)
    csems = (csem0, csem1)
    osems = (osem0, osem1)

    def _reg(buf):
        return buf.at[pl.ds(sid * CH, CH)]

    G = [None] * NCHUNK
    C = [None] * NCHUNK
    O = [None] * NCHUNK
    G[0] = pltpu.async_copy(
        table_hbm.at[idx_v.at[pl.ds(0, CH)]], vbufs[0], gsems[0])
    G[1] = pltpu.async_copy(
        table_hbm.at[idx_v.at[pl.ds(CH, CH)]], vbufs[1], gsems[1])
    for i in range(NCHUNK):
        b = i % 2
        G[i].wait()
        if i >= 2:
            O[i - 2].wait()
        C[i] = pltpu.async_copy(vbufs[b], _reg(sbufs[b]), csems[b])
        C[i].wait()
        O[i] = pltpu.async_copy(
            _reg(sbufs[b]), out_hbm.at[pl.ds(base + i * CH, CH)], osems[b])
        if i + 2 < NCHUNK:
            G[i + 2] = pltpu.async_copy(
                table_hbm.at[idx_v.at[pl.ds((i + 2) * CH, CH)]],
                vbufs[b], gsems[b])
    O[NCHUNK - 2].wait()
    O[NCHUNK - 1].wait()


@jax.jit
def _gather(table, idx):
    k = functools.partial(
        pl.kernel,
        mesh=plsc.VectorSubcoreMesh(core_axis_name="c", subcore_axis_name="s"),
        out_type=jax.ShapeDtypeStruct((B, D), jnp.float32),
        scratch_types=[
            pltpu.VMEM((B_PER_W,), jnp.int32),
            pltpu.VMEM((CH, D), jnp.float32),
            pltpu.VMEM((CH, D), jnp.float32),
            pltpu.MemorySpace.VMEM_SHARED((NS * CH, D), jnp.float32),
            pltpu.MemorySpace.VMEM_SHARED((NS * CH, D), jnp.float32),
            pltpu.SemaphoreType.DMA,
            pltpu.SemaphoreType.DMA,
            pltpu.SemaphoreType.DMA,
            pltpu.SemaphoreType.DMA,
            pltpu.SemaphoreType.DMA,
            pltpu.SemaphoreType.DMA,
        ],
    )(_gather_kernel)
    return k(table, idx)


def kernel(hidden_state, word_indices):
    table = hidden_state.reshape(V, D)
    idx = word_indices.astype(jnp.int32)
    out = _gather(table, idx)
    return out.reshape(1, B, D)
